# baseline (speedup 1.0000x reference)
import functools

import jax
import jax.numpy as jnp
import numpy as np
from jax import lax
from jax.experimental import pallas as pl
from jax.experimental.pallas import tpu as pltpu
from jax.experimental.pallas import tpu_sc as plsc

N = 4194304
H, W, C = 480, 640, 2
N_EVENT, N_PROP = 10, 10
TOTAL = N_EVENT + N_PROP

NBINS_USED = N_EVENT * C * H * W
NBINS_TOTAL = TOTAL * C * H * W
NCHUNK = 4
CHUNK = NBINS_USED // NCHUNK
PAD_SPREAD = 32768
SPMEM_WORDS = 1572864
NS = 16
NC = 2
WSZ = 8192
EBLK = N // NS
WIN_PER_TILE = EBLK // WSZ
ZERO_STRIDE = SPMEM_WORDS // NS
DRAIN = CHUNK // NS
TAIL = NBINS_TOTAL - NBINS_USED
TAIL_PER_TILE = TAIL // (NC * NS)


def _minmax_body(t_ref, mn_ref, mx_ref):
    i = pl.program_id(0)
    bmn = jnp.min(t_ref[...])
    bmx = jnp.max(t_ref[...])

    @pl.when(i == 0)
    def _():
        mn_ref[0, 0] = bmn
        mx_ref[0, 0] = bmx

    @pl.when(i > 0)
    def _():
        mn_ref[0, 0] = jnp.minimum(mn_ref[0, 0], bmn)
        mx_ref[0, 0] = jnp.maximum(mx_ref[0, 0], bmx)


def _linearize_body(mn_ref, mx_ref, x_ref, y_ref, p_ref, t_ref, pad_ref,
                    out_ref):
    tmin = mn_ref[0, 0]
    tmax = mx_ref[0, 0]
    ok = tmax > tmin
    denom = jnp.where(ok, tmax - tmin, jnp.float32(1.0))
    scale = jnp.float32(N_EVENT - 1e-06)
    t = t_ref[...]
    t_norm = jnp.where(ok, (t - tmin) / denom * scale, jnp.zeros_like(t))
    ti = jnp.clip(t_norm.astype(jnp.int32), 0, N_EVENT - 1)
    idx = ((ti * C + p_ref[...]) * H + y_ref[...]) * W + x_ref[...]

    cid = ((idx >= CHUNK).astype(jnp.int32)
           + (idx >= 2 * CHUNK).astype(jnp.int32)
           + (idx >= 3 * CHUNK).astype(jnp.int32))
    pad = pad_ref[...]
    for b in range(NCHUNK):
        out_ref[pl.ds(b * EBLK, EBLK)] = jnp.where(
            cid == b, idx - b * CHUNK, pad)


def _sc_scatter_body(idx_hbm, ones_hbm, zeros_hbm, out_hbm,
                     spmem, idxv0, idxv1, onesv, zerov, sem, tail_sem):
    idxv = (idxv0, idxv1)
    cid = lax.axis_index("c")
    sid = lax.axis_index("s")
    wid = cid * NS + sid

    pltpu.sync_copy(ones_hbm, onesv)
    pltpu.sync_copy(zeros_hbm.at[pl.ds(0, WSZ)], zerov)

    tail_base = NBINS_USED + wid * TAIL_PER_TILE
    n_full = TAIL_PER_TILE // WSZ
    rem = TAIL_PER_TILE - n_full * WSZ
    tail_cps = []
    for k in range(n_full):
        tail_cps.append(pltpu.async_copy(
            zerov, out_hbm.at[pl.ds(tail_base + k * WSZ, WSZ)], tail_sem))
    if rem:
        tail_cps.append(pltpu.async_copy(
            zerov.at[pl.ds(0, rem)],
            out_hbm.at[pl.ds(tail_base + n_full * WSZ, rem)], tail_sem))

    for r in range(NCHUNK // NC):
        b = NC * r + cid

        pltpu.sync_copy(zeros_hbm, spmem.at[pl.ds(sid * DRAIN, DRAIN)])
        plsc.subcore_barrier()

        base = sid * (NCHUNK * EBLK) + b * EBLK
        cp = pltpu.async_copy(idx_hbm.at[pl.ds(base, WSZ)], idxv[0], sem)
        for wi in range(WIN_PER_TILE):
            cp.wait()
            if wi + 1 < WIN_PER_TILE:
                cp = pltpu.async_copy(
                    idx_hbm.at[pl.ds(base + (wi + 1) * WSZ, WSZ)],
                    idxv[(wi + 1) % 2], sem)
            pltpu.sync_copy(onesv, spmem.at[idxv[wi % 2]], add=True)
        plsc.subcore_barrier()

        pltpu.sync_copy(
            spmem.at[pl.ds(sid * DRAIN, DRAIN)],
            out_hbm.at[pl.ds(b * CHUNK + sid * DRAIN, DRAIN)])

    for cp_t in tail_cps:
        cp_t.wait()


@jax.jit
def kernel(x, y, p, t):
    x = x.astype(jnp.int32)
    y = y.astype(jnp.int32)
    p = p.astype(jnp.int32)
    t = t.astype(jnp.float32)

    grid = N // EBLK
    blk = (EBLK,)

    mblk = 524288
    tmin, tmax = pl.pallas_call(
        _minmax_body,
        grid=(N // mblk,),
        in_specs=[pl.BlockSpec((mblk,), lambda i: (i,))],
        out_specs=[pl.BlockSpec(memory_space=pltpu.MemorySpace.SMEM),
                   pl.BlockSpec(memory_space=pltpu.MemorySpace.SMEM)],
        out_shape=[jax.ShapeDtypeStruct((1, 1), jnp.float32),
                   jax.ShapeDtypeStruct((1, 1), jnp.float32)],
    )(t)

    padv = (jnp.arange(EBLK, dtype=jnp.int32) % PAD_SPREAD) + CHUNK

    idx_all = pl.pallas_call(
        _linearize_body,
        grid=(grid,),
        in_specs=[pl.BlockSpec(memory_space=pltpu.MemorySpace.SMEM),
                  pl.BlockSpec(memory_space=pltpu.MemorySpace.SMEM),
                  pl.BlockSpec(blk, lambda i: (i,)),
                  pl.BlockSpec(blk, lambda i: (i,)),
                  pl.BlockSpec(blk, lambda i: (i,)),
                  pl.BlockSpec(blk, lambda i: (i,)),
                  pl.BlockSpec(blk, lambda i: (0,))],
        out_specs=pl.BlockSpec((NCHUNK * EBLK,), lambda i: (i,)),
        out_shape=jax.ShapeDtypeStruct((NCHUNK * N,), jnp.int32),
    )(tmin, tmax, x, y, p, t, padv)

    ones = jnp.ones((WSZ,), jnp.float32)
    zeros = jnp.zeros((DRAIN,), jnp.float32)

    sc = functools.partial(
        pl.kernel,
        out_type=jax.ShapeDtypeStruct((NBINS_TOTAL,), jnp.float32),
        mesh=plsc.VectorSubcoreMesh(core_axis_name="c", subcore_axis_name="s"),
        scratch_types=[
            pltpu.VMEM_SHARED((SPMEM_WORDS,), jnp.float32),
            pltpu.VMEM((WSZ,), jnp.int32),
            pltpu.VMEM((WSZ,), jnp.int32),
            pltpu.VMEM((WSZ,), jnp.float32),
            pltpu.VMEM((WSZ,), jnp.float32),
            pltpu.SemaphoreType.DMA,
            pltpu.SemaphoreType.DMA,
        ],
    )(_sc_scatter_body)

    buf = sc(idx_all, ones, zeros)
    return buf.reshape(TOTAL, C, H, W)


# speedup vs baseline: 1.0170x; 1.0170x over previous
import functools

import jax
import jax.numpy as jnp
import numpy as np
from jax import lax
from jax.experimental import pallas as pl
from jax.experimental.pallas import tpu as pltpu
from jax.experimental.pallas import tpu_sc as plsc

N = 4194304
H, W, C = 480, 640, 2
N_EVENT, N_PROP = 10, 10
TOTAL = N_EVENT + N_PROP

NBINS_USED = N_EVENT * C * H * W
NBINS_TOTAL = TOTAL * C * H * W
NCHUNK = 4
CHUNK = NBINS_USED // NCHUNK
PAD_SPREAD = 32768
SPMEM_WORDS = 1572864
NS = 16
NC = 2
WSZ = 8192
EBLK = N // NS
WIN_PER_TILE = EBLK // WSZ
ZERO_STRIDE = SPMEM_WORDS // NS
DRAIN = CHUNK // NS
TAIL = NBINS_TOTAL - NBINS_USED
TAIL_PER_TILE = TAIL // (NC * NS)


def _minmax_body(t_ref, mn_ref, mx_ref):
    i = pl.program_id(0)
    bmn = jnp.min(t_ref[...])
    bmx = jnp.max(t_ref[...])

    @pl.when(i == 0)
    def _():
        mn_ref[0, 0] = bmn
        mx_ref[0, 0] = bmx

    @pl.when(i > 0)
    def _():
        mn_ref[0, 0] = jnp.minimum(mn_ref[0, 0], bmn)
        mx_ref[0, 0] = jnp.maximum(mx_ref[0, 0], bmx)


def _linearize_body(mn_ref, mx_ref, x_ref, y_ref, p_ref, t_ref, pad_ref,
                    out_ref):
    tmin = mn_ref[0, 0]
    tmax = mx_ref[0, 0]
    ok = tmax > tmin
    denom = jnp.where(ok, tmax - tmin, jnp.float32(1.0))
    scale = jnp.float32(N_EVENT - 1e-06)
    t = t_ref[...]
    t_norm = jnp.where(ok, (t - tmin) / denom * scale, jnp.zeros_like(t))
    ti = jnp.clip(t_norm.astype(jnp.int32), 0, N_EVENT - 1)
    idx = ((ti * C + p_ref[...]) * H + y_ref[...]) * W + x_ref[...]

    cid = ((idx >= CHUNK).astype(jnp.int32)
           + (idx >= 2 * CHUNK).astype(jnp.int32)
           + (idx >= 3 * CHUNK).astype(jnp.int32))
    pad = pad_ref[...]
    for b in range(NCHUNK):
        out_ref[pl.ds(b * EBLK, EBLK)] = jnp.where(
            cid == b, idx - b * CHUNK, pad)


def _sc_scatter_body(idx_hbm, ones_hbm, zeros_hbm, out_hbm,
                     spmem, idxv0, idxv1, onesv, zerov, sem, tail_sem):
    idxv = (idxv0, idxv1)
    cid = lax.axis_index("c")
    sid = lax.axis_index("s")
    wid = cid * NS + sid

    pltpu.sync_copy(ones_hbm, onesv)
    pltpu.sync_copy(zeros_hbm, zerov)

    tail_base = NBINS_USED + wid * TAIL_PER_TILE
    n_full = TAIL_PER_TILE // WSZ
    rem = TAIL_PER_TILE - n_full * WSZ
    tail_cps = []
    for k in range(n_full):
        tail_cps.append(pltpu.async_copy(
            zerov, out_hbm.at[pl.ds(tail_base + k * WSZ, WSZ)], tail_sem))
    if rem:
        tail_cps.append(pltpu.async_copy(
            zerov.at[pl.ds(0, rem)],
            out_hbm.at[pl.ds(tail_base + n_full * WSZ, rem)], tail_sem))

    for r in range(NCHUNK // NC):
        b = NC * r + cid

        zfull = DRAIN // WSZ
        zrem = DRAIN - zfull * WSZ
        for k in range(zfull):
            pltpu.sync_copy(
                zerov, spmem.at[pl.ds(sid * DRAIN + k * WSZ, WSZ)])
        if zrem:
            pltpu.sync_copy(
                zerov.at[pl.ds(0, zrem)],
                spmem.at[pl.ds(sid * DRAIN + zfull * WSZ, zrem)])
        plsc.subcore_barrier()

        base = sid * (NCHUNK * EBLK) + b * EBLK
        cp = pltpu.async_copy(idx_hbm.at[pl.ds(base, WSZ)], idxv[0], sem)
        for wi in range(WIN_PER_TILE):
            cp.wait()
            if wi + 1 < WIN_PER_TILE:
                cp = pltpu.async_copy(
                    idx_hbm.at[pl.ds(base + (wi + 1) * WSZ, WSZ)],
                    idxv[(wi + 1) % 2], sem)
            pltpu.sync_copy(onesv, spmem.at[idxv[wi % 2]], add=True)
        plsc.subcore_barrier()

        pltpu.sync_copy(
            spmem.at[pl.ds(sid * DRAIN, DRAIN)],
            out_hbm.at[pl.ds(b * CHUNK + sid * DRAIN, DRAIN)])

    for cp_t in tail_cps:
        cp_t.wait()


@jax.jit
def kernel(x, y, p, t):
    x = x.astype(jnp.int32)
    y = y.astype(jnp.int32)
    p = p.astype(jnp.int32)
    t = t.astype(jnp.float32)

    grid = N // EBLK
    blk = (EBLK,)

    mblk = 524288
    tmin, tmax = pl.pallas_call(
        _minmax_body,
        grid=(N // mblk,),
        in_specs=[pl.BlockSpec((mblk,), lambda i: (i,))],
        out_specs=[pl.BlockSpec(memory_space=pltpu.MemorySpace.SMEM),
                   pl.BlockSpec(memory_space=pltpu.MemorySpace.SMEM)],
        out_shape=[jax.ShapeDtypeStruct((1, 1), jnp.float32),
                   jax.ShapeDtypeStruct((1, 1), jnp.float32)],
    )(t)

    padv = (jnp.arange(EBLK, dtype=jnp.int32) % PAD_SPREAD) + CHUNK

    idx_all = pl.pallas_call(
        _linearize_body,
        grid=(grid,),
        in_specs=[pl.BlockSpec(memory_space=pltpu.MemorySpace.SMEM),
                  pl.BlockSpec(memory_space=pltpu.MemorySpace.SMEM),
                  pl.BlockSpec(blk, lambda i: (i,)),
                  pl.BlockSpec(blk, lambda i: (i,)),
                  pl.BlockSpec(blk, lambda i: (i,)),
                  pl.BlockSpec(blk, lambda i: (i,)),
                  pl.BlockSpec(blk, lambda i: (0,))],
        out_specs=pl.BlockSpec((NCHUNK * EBLK,), lambda i: (i,)),
        out_shape=jax.ShapeDtypeStruct((NCHUNK * N,), jnp.int32),
    )(tmin, tmax, x, y, p, t, padv)

    ones = jnp.ones((WSZ,), jnp.float32)
    zeros = jnp.zeros((WSZ,), jnp.float32)

    sc = functools.partial(
        pl.kernel,
        out_type=jax.ShapeDtypeStruct((NBINS_TOTAL,), jnp.float32),
        mesh=plsc.VectorSubcoreMesh(core_axis_name="c", subcore_axis_name="s"),
        scratch_types=[
            pltpu.VMEM_SHARED((SPMEM_WORDS,), jnp.float32),
            pltpu.VMEM((WSZ,), jnp.int32),
            pltpu.VMEM((WSZ,), jnp.int32),
            pltpu.VMEM((WSZ,), jnp.float32),
            pltpu.VMEM((WSZ,), jnp.float32),
            pltpu.SemaphoreType.DMA,
            pltpu.SemaphoreType.DMA,
        ],
    )(_sc_scatter_body)

    buf = sc(idx_all, ones, zeros)
    return buf.reshape(TOTAL, C, H, W)


# speedup vs baseline: 1.0288x; 1.0116x over previous
import functools

import jax
import jax.numpy as jnp
import numpy as np
from jax import lax
from jax.experimental import pallas as pl
from jax.experimental.pallas import tpu as pltpu
from jax.experimental.pallas import tpu_sc as plsc

N = 4194304
H, W, C = 480, 640, 2
N_EVENT, N_PROP = 10, 10
TOTAL = N_EVENT + N_PROP

NBINS_USED = N_EVENT * C * H * W
NBINS_TOTAL = TOTAL * C * H * W
NCHUNK = 4
CHUNK = NBINS_USED // NCHUNK
PAD_SPREAD = 32768
SPMEM_WORDS = 1572864
NS = 16
NC = 2
WSZ = 4096
EBLK = N // NS
WIN_PER_TILE = EBLK // WSZ
ZERO_STRIDE = SPMEM_WORDS // NS
DRAIN = CHUNK // NS
TAIL = NBINS_TOTAL - NBINS_USED
TAIL_PER_TILE = TAIL // (NC * NS)


def _minmax_body(t_ref, mn_ref, mx_ref):
    i = pl.program_id(0)
    bmn = jnp.min(t_ref[...])
    bmx = jnp.max(t_ref[...])

    @pl.when(i == 0)
    def _():
        mn_ref[0, 0] = bmn
        mx_ref[0, 0] = bmx

    @pl.when(i > 0)
    def _():
        mn_ref[0, 0] = jnp.minimum(mn_ref[0, 0], bmn)
        mx_ref[0, 0] = jnp.maximum(mx_ref[0, 0], bmx)


def _linearize_body(mn_ref, mx_ref, x_ref, y_ref, p_ref, t_ref, pad_ref,
                    out_ref):
    tmin = mn_ref[0, 0]
    tmax = mx_ref[0, 0]
    ok = tmax > tmin
    denom = jnp.where(ok, tmax - tmin, jnp.float32(1.0))
    scale = jnp.float32(N_EVENT - 1e-06)
    t = t_ref[...]
    t_norm = jnp.where(ok, (t - tmin) / denom * scale, jnp.zeros_like(t))
    ti = jnp.clip(t_norm.astype(jnp.int32), 0, N_EVENT - 1)
    idx = ((ti * C + p_ref[...]) * H + y_ref[...]) * W + x_ref[...]

    cid = ((idx >= CHUNK).astype(jnp.int32)
           + (idx >= 2 * CHUNK).astype(jnp.int32)
           + (idx >= 3 * CHUNK).astype(jnp.int32))
    pad = pad_ref[...]
    for b in range(NCHUNK):
        out_ref[pl.ds(b * EBLK, EBLK)] = jnp.where(
            cid == b, idx - b * CHUNK, pad)


def _sc_scatter_body(idx_hbm, ones_hbm, zeros_hbm, out_hbm,
                     spmem, idxv0, idxv1, idxv2, onesv, zerov,
                     sem, scat_sem, tail_sem):
    idxv = (idxv0, idxv1, idxv2)
    cid = lax.axis_index("c")
    sid = lax.axis_index("s")
    wid = cid * NS + sid

    pltpu.sync_copy(ones_hbm, onesv)
    pltpu.sync_copy(zeros_hbm, zerov)

    tail_base = NBINS_USED + wid * TAIL_PER_TILE
    n_full = TAIL_PER_TILE // WSZ
    rem = TAIL_PER_TILE - n_full * WSZ
    tail_cps = []
    for k in range(n_full):
        tail_cps.append(pltpu.async_copy(
            zerov, out_hbm.at[pl.ds(tail_base + k * WSZ, WSZ)], tail_sem))
    if rem:
        tail_cps.append(pltpu.async_copy(
            zerov.at[pl.ds(0, rem)],
            out_hbm.at[pl.ds(tail_base + n_full * WSZ, rem)], tail_sem))

    for r in range(NCHUNK // NC):
        b = NC * r + cid

        zfull = DRAIN // WSZ
        zrem = DRAIN - zfull * WSZ
        for k in range(zfull):
            pltpu.sync_copy(
                zerov, spmem.at[pl.ds(sid * DRAIN + k * WSZ, WSZ)])
        if zrem:
            pltpu.sync_copy(
                zerov.at[pl.ds(0, zrem)],
                spmem.at[pl.ds(sid * DRAIN + zfull * WSZ, zrem)])
        plsc.subcore_barrier()

        base = sid * (NCHUNK * EBLK) + b * EBLK
        cp = pltpu.async_copy(idx_hbm.at[pl.ds(base, WSZ)], idxv[0], sem)
        scats = [None, None, None]
        for wi in range(WIN_PER_TILE):
            cp.wait()
            if scats[(wi + 1) % 3] is not None:
                scats[(wi + 1) % 3].wait()
                scats[(wi + 1) % 3] = None
            if wi + 1 < WIN_PER_TILE:
                cp = pltpu.async_copy(
                    idx_hbm.at[pl.ds(base + (wi + 1) * WSZ, WSZ)],
                    idxv[(wi + 1) % 3], sem)
            scats[wi % 3] = pltpu.async_copy(
                onesv, spmem.at[idxv[wi % 3]], scat_sem, add=True)
        for s in scats:
            if s is not None:
                s.wait()
        plsc.subcore_barrier()

        pltpu.sync_copy(
            spmem.at[pl.ds(sid * DRAIN, DRAIN)],
            out_hbm.at[pl.ds(b * CHUNK + sid * DRAIN, DRAIN)])

    for cp_t in tail_cps:
        cp_t.wait()


@jax.jit
def kernel(x, y, p, t):
    x = x.astype(jnp.int32)
    y = y.astype(jnp.int32)
    p = p.astype(jnp.int32)
    t = t.astype(jnp.float32)

    grid = N // EBLK
    blk = (EBLK,)

    mblk = 524288
    tmin, tmax = pl.pallas_call(
        _minmax_body,
        grid=(N // mblk,),
        in_specs=[pl.BlockSpec((mblk,), lambda i: (i,))],
        out_specs=[pl.BlockSpec(memory_space=pltpu.MemorySpace.SMEM),
                   pl.BlockSpec(memory_space=pltpu.MemorySpace.SMEM)],
        out_shape=[jax.ShapeDtypeStruct((1, 1), jnp.float32),
                   jax.ShapeDtypeStruct((1, 1), jnp.float32)],
    )(t)

    padv = (jnp.arange(EBLK, dtype=jnp.int32) % PAD_SPREAD) + CHUNK

    idx_all = pl.pallas_call(
        _linearize_body,
        grid=(grid,),
        in_specs=[pl.BlockSpec(memory_space=pltpu.MemorySpace.SMEM),
                  pl.BlockSpec(memory_space=pltpu.MemorySpace.SMEM),
                  pl.BlockSpec(blk, lambda i: (i,)),
                  pl.BlockSpec(blk, lambda i: (i,)),
                  pl.BlockSpec(blk, lambda i: (i,)),
                  pl.BlockSpec(blk, lambda i: (i,)),
                  pl.BlockSpec(blk, lambda i: (0,))],
        out_specs=pl.BlockSpec((NCHUNK * EBLK,), lambda i: (i,)),
        out_shape=jax.ShapeDtypeStruct((NCHUNK * N,), jnp.int32),
    )(tmin, tmax, x, y, p, t, padv)

    ones = jnp.ones((WSZ,), jnp.float32)
    zeros = jnp.zeros((WSZ,), jnp.float32)

    sc = functools.partial(
        pl.kernel,
        out_type=jax.ShapeDtypeStruct((NBINS_TOTAL,), jnp.float32),
        mesh=plsc.VectorSubcoreMesh(core_axis_name="c", subcore_axis_name="s"),
        scratch_types=[
            pltpu.VMEM_SHARED((SPMEM_WORDS,), jnp.float32),
            pltpu.VMEM((WSZ,), jnp.int32),
            pltpu.VMEM((WSZ,), jnp.int32),
            pltpu.VMEM((WSZ,), jnp.int32),
            pltpu.VMEM((WSZ,), jnp.float32),
            pltpu.VMEM((WSZ,), jnp.float32),
            pltpu.SemaphoreType.DMA,
            pltpu.SemaphoreType.DMA,
            pltpu.SemaphoreType.DMA,
        ],
    )(_sc_scatter_body)

    buf = sc(idx_all, ones, zeros)
    return buf.reshape(TOTAL, C, H, W)


# speedup vs baseline: 1.0296x; 1.0008x over previous
import functools

import jax
import jax.numpy as jnp
from jax import lax
from jax.experimental import pallas as pl
from jax.experimental.pallas import tpu as pltpu
from jax.experimental.pallas import tpu_sc as plsc

N = 4194304
H, W, C = 480, 640, 2
N_EVENT, N_PROP = 10, 10
TOTAL = N_EVENT + N_PROP

NBINS_USED = N_EVENT * C * H * W
NBINS_TOTAL = TOTAL * C * H * W
NCHUNK = 4
CHUNK = NBINS_USED // NCHUNK
PAD_SPREAD = 32768
SPMEM_WORDS = 1572864
NS = 16
NC = 2
WSZ = 4096
EBLK = N // NS
WIN_PER_TILE = EBLK // WSZ
DRAIN = CHUNK // NS
TAIL = NBINS_TOTAL - NBINS_USED
TAIL_PER_TILE = TAIL // (NC * NS)


def _minmax_body(t_ref, mn_ref, mx_ref):
    i = pl.program_id(0)
    bmn = jnp.min(t_ref[...])
    bmx = jnp.max(t_ref[...])

    @pl.when(i == 0)
    def _():
        mn_ref[0, 0] = bmn
        mx_ref[0, 0] = bmx

    @pl.when(i > 0)
    def _():
        mn_ref[0, 0] = jnp.minimum(mn_ref[0, 0], bmn)
        mx_ref[0, 0] = jnp.maximum(mx_ref[0, 0], bmx)


def _linearize_body(mn_ref, mx_ref, x_ref, y_ref, p_ref, t_ref, pad_ref,
                    out_ref):
    tmin = mn_ref[0, 0]
    tmax = mx_ref[0, 0]
    ok = tmax > tmin
    denom = jnp.where(ok, tmax - tmin, jnp.float32(1.0))
    scale = jnp.float32(N_EVENT - 1e-06)
    t = t_ref[...]
    t_norm = jnp.where(ok, (t - tmin) / denom * scale, jnp.zeros_like(t))
    ti = jnp.clip(t_norm.astype(jnp.int32), 0, N_EVENT - 1)
    idx = ((ti * C + p_ref[...]) * H + y_ref[...]) * W + x_ref[...]

    cid = ((idx >= CHUNK).astype(jnp.int32)
           + (idx >= 2 * CHUNK).astype(jnp.int32)
           + (idx >= 3 * CHUNK).astype(jnp.int32))
    pad = pad_ref[...]
    for b in range(NCHUNK):
        out_ref[pl.ds(b * EBLK, EBLK)] = jnp.where(
            cid == b, idx - b * CHUNK, pad)


def _sc_scatter_body(idx_hbm, ones_hbm, zeros_hbm, out_hbm,
                     spmem, idxv0, idxv1, idxv2, onesv, zerov,
                     sem, scat_sem, tail_sem):
    idxv = (idxv0, idxv1, idxv2)
    cid = lax.axis_index("c")
    sid = lax.axis_index("s")
    wid = cid * NS + sid

    pltpu.sync_copy(ones_hbm, onesv)
    pltpu.sync_copy(zeros_hbm, zerov)

    tail_base = NBINS_USED + wid * TAIL_PER_TILE
    n_full = TAIL_PER_TILE // WSZ
    rem = TAIL_PER_TILE - n_full * WSZ
    tail_cps = []
    for k in range(n_full):
        tail_cps.append(pltpu.async_copy(
            zerov, out_hbm.at[pl.ds(tail_base + k * WSZ, WSZ)], tail_sem))
    if rem:
        tail_cps.append(pltpu.async_copy(
            zerov.at[pl.ds(0, rem)],
            out_hbm.at[pl.ds(tail_base + n_full * WSZ, rem)], tail_sem))

    for r in range(NCHUNK // NC):
        b = NC * r + cid

        zfull = DRAIN // WSZ
        zrem = DRAIN - zfull * WSZ
        for k in range(zfull):
            pltpu.sync_copy(
                zerov, spmem.at[pl.ds(sid * DRAIN + k * WSZ, WSZ)])
        if zrem:
            pltpu.sync_copy(
                zerov.at[pl.ds(0, zrem)],
                spmem.at[pl.ds(sid * DRAIN + zfull * WSZ, zrem)])
        plsc.subcore_barrier()

        base = sid * (NCHUNK * EBLK) + b * EBLK
        cp = pltpu.async_copy(idx_hbm.at[pl.ds(base, WSZ)], idxv[0], sem)
        scats = [None, None, None]
        for wi in range(WIN_PER_TILE):
            cp.wait()
            if scats[(wi + 1) % 3] is not None:
                scats[(wi + 1) % 3].wait()
                scats[(wi + 1) % 3] = None
            if wi + 1 < WIN_PER_TILE:
                cp = pltpu.async_copy(
                    idx_hbm.at[pl.ds(base + (wi + 1) * WSZ, WSZ)],
                    idxv[(wi + 1) % 3], sem)
            scats[wi % 3] = pltpu.async_copy(
                onesv, spmem.at[idxv[wi % 3]], scat_sem, add=True)
        for s in scats:
            if s is not None:
                s.wait()
        plsc.subcore_barrier()

        pltpu.sync_copy(
            spmem.at[pl.ds(sid * DRAIN, DRAIN)],
            out_hbm.at[pl.ds(b * CHUNK + sid * DRAIN, DRAIN)])

    for cp_t in tail_cps:
        cp_t.wait()


@jax.jit
def kernel(x, y, p, t):
    x = x.astype(jnp.int32)
    y = y.astype(jnp.int32)
    p = p.astype(jnp.int32)
    t = t.astype(jnp.float32)

    grid = N // EBLK
    blk = (EBLK,)

    mblk = 1048576
    tmin, tmax = pl.pallas_call(
        _minmax_body,
        grid=(N // mblk,),
        in_specs=[pl.BlockSpec((mblk,), lambda i: (i,))],
        out_specs=[pl.BlockSpec(memory_space=pltpu.MemorySpace.SMEM),
                   pl.BlockSpec(memory_space=pltpu.MemorySpace.SMEM)],
        out_shape=[jax.ShapeDtypeStruct((1, 1), jnp.float32),
                   jax.ShapeDtypeStruct((1, 1), jnp.float32)],
    )(t)

    padv = (jnp.arange(EBLK, dtype=jnp.int32) % PAD_SPREAD) + CHUNK

    idx_all = pl.pallas_call(
        _linearize_body,
        grid=(grid,),
        in_specs=[pl.BlockSpec(memory_space=pltpu.MemorySpace.SMEM),
                  pl.BlockSpec(memory_space=pltpu.MemorySpace.SMEM),
                  pl.BlockSpec(blk, lambda i: (i,)),
                  pl.BlockSpec(blk, lambda i: (i,)),
                  pl.BlockSpec(blk, lambda i: (i,)),
                  pl.BlockSpec(blk, lambda i: (i,)),
                  pl.BlockSpec(blk, lambda i: (0,))],
        out_specs=pl.BlockSpec((NCHUNK * EBLK,), lambda i: (i,)),
        out_shape=jax.ShapeDtypeStruct((NCHUNK * N,), jnp.int32),
    )(tmin, tmax, x, y, p, t, padv)

    ones = jnp.ones((WSZ,), jnp.float32)
    zeros = jnp.zeros((WSZ,), jnp.float32)

    sc = functools.partial(
        pl.kernel,
        out_type=jax.ShapeDtypeStruct((NBINS_TOTAL,), jnp.float32),
        mesh=plsc.VectorSubcoreMesh(core_axis_name="c", subcore_axis_name="s"),
        scratch_types=[
            pltpu.VMEM_SHARED((SPMEM_WORDS,), jnp.float32),
            pltpu.VMEM((WSZ,), jnp.int32),
            pltpu.VMEM((WSZ,), jnp.int32),
            pltpu.VMEM((WSZ,), jnp.int32),
            pltpu.VMEM((WSZ,), jnp.float32),
            pltpu.VMEM((WSZ,), jnp.float32),
            pltpu.SemaphoreType.DMA,
            pltpu.SemaphoreType.DMA,
            pltpu.SemaphoreType.DMA,
        ],
    )(_sc_scatter_body)

    buf = sc(idx_all, ones, zeros)
    return buf.reshape(TOTAL, C, H, W)
